# Initial kernel scaffold; baseline (speedup 1.0000x reference)
import jax
import jax.numpy as jnp
from jax.experimental import pallas as pl


def kernel(x, pos, wq, wk, wv, pw1, pb1, pw2, pb2, aw1, ab1, aw2, ab2, fw, fb):
    raise NotImplementedError("write your pallas kernel here")


# speedup vs baseline: 11.5732x; 11.5732x over previous
import functools

import jax
import jax.numpy as jnp
from jax import lax
from jax.experimental import pallas as pl
from jax.experimental.pallas import tpu as pltpu
from jax.experimental.pallas import tpu_sc as plsc

K = 16
TW = 128
RB = 128
BN = 128
CHUNK = 128
NBUF = 4


def _knn_body(pos_ref, post_ref, out_ref):
    b = pl.program_id(0)
    n_total = post_ref.shape[2]
    pc = pos_ref[0]
    pt = post_ref[0]
    acc = jnp.zeros((pc.shape[0], n_total), jnp.float32)
    for d in range(3):
        diff = pc[:, d:d + 1] - pt[d:d + 1, :]
        acc = acc + diff * diff
    iota = lax.broadcasted_iota(jnp.int32, acc.shape, 1)
    cols = []
    for _ in range(K):
        m = jnp.min(acc, axis=1, keepdims=True)
        cand = jnp.where(acc <= m, iota, n_total)
        idx = jnp.min(cand, axis=1, keepdims=True)
        cols.append(idx)
        acc = jnp.where(iota == idx, jnp.inf, acc)
    out_ref[0] = jnp.concatenate(cols, axis=1) + b * n_total


def _knn(pos, post):
    b, n, _ = pos.shape
    return pl.pallas_call(
        _knn_body,
        grid=(b, n // RB),
        in_specs=[
            pl.BlockSpec((1, RB, 3), lambda bi, i: (bi, i, 0)),
            pl.BlockSpec((1, 3, n), lambda bi, i: (bi, 0, 0)),
        ],
        out_specs=pl.BlockSpec((1, RB, K), lambda bi, i: (bi, i, 0)),
        out_shape=jax.ShapeDtypeStruct((b, n, K), jnp.int32),
    )(pos, post)


def _sc_gather(tbl2, idx3, tot):
    info = plsc.get_sparse_core_info()
    nc, ns = info.num_cores, info.num_subcores
    nw = nc * ns
    cpw = idx3.shape[1]
    mesh = plsc.VectorSubcoreMesh(core_axis_name="c", subcore_axis_name="s")

    @functools.partial(
        pl.kernel,
        mesh=mesh,
        out_type=jax.ShapeDtypeStruct((tot, TW), jnp.float32),
        scratch_types=(
            [pltpu.VMEM((cpw, CHUNK), jnp.int32)]
            + [pltpu.VMEM((CHUNK, TW), jnp.float32) for _ in range(NBUF)]
            + [pltpu.SemaphoreType.DMA, pltpu.SemaphoreType.DMA]
        ),
    )
    def k(tbl_hbm, idx_hbm, out_hbm, idx_v, *rest):
        bufs = rest[:NBUF]
        gsem, ssem = rest[NBUF], rest[NBUF + 1]
        wid = lax.axis_index("s") * nc + lax.axis_index("c")
        pltpu.sync_copy(idx_hbm.at[wid], idx_v)
        for r in range(cpw // NBUF):
            gets = []
            for j in range(NBUF):
                c = r * NBUF + j
                gets.append(
                    pltpu.async_copy(tbl_hbm.at[idx_v.at[c]], bufs[j], gsem))
            for g in gets:
                g.wait()
            puts = []
            for j in range(NBUF):
                c = r * NBUF + j
                base = (wid * cpw + c) * CHUNK
                puts.append(
                    pltpu.async_copy(bufs[j], out_hbm.at[pl.ds(base, CHUNK)],
                                     ssem))
            for p in puts:
                p.wait()

    return k(tbl2, idx3)


def _dot(a, b):
    return jnp.dot(a, b, precision=lax.Precision.HIGHEST,
                   preferred_element_type=jnp.float32)


def _attn_body(tbl_ref, g_ref, wq_ref, wkv_ref, pw1_ref, pb1_ref, pw2_ref,
               pb2_ref, aw1_ref, ab1_ref, aw2_ref, ab2_ref, fw_ref, fb_ref,
               out_ref):
    tbl = tbl_ref[0]
    g = g_ref[0]
    xc = tbl[:, 0:64]
    pic = tbl[:, 64:80]
    xg = g[:, 0:64]
    pjg = g[:, 64:80]

    q = _dot(xc, wq_ref[...])
    kv = _dot(xg, wkv_ref[...])
    rel = pic[:, None, :] - pjg.reshape(BN, K, 16)
    pe = _dot(jax.nn.relu(_dot(rel.reshape(BN * K, 16), pw1_ref[...])
                          + pb1_ref[...]), pw2_ref[...]) + pb2_ref[...]
    h = (q[:, None, :] - kv[:, 0:64].reshape(BN, K, 64)
         + pe.reshape(BN, K, 64))
    a = _dot(jax.nn.relu(_dot(h.reshape(BN * K, 64), aw1_ref[...])
                         + ab1_ref[...]), aw2_ref[...]) + ab2_ref[...]
    s = a.reshape(BN, K, 64) * 0.125
    s = s - jnp.max(s, axis=1, keepdims=True)
    e = jnp.exp(s)
    w = e / jnp.sum(e, axis=1, keepdims=True)
    vpe = kv[:, 64:128].reshape(BN, K, 64) + pe.reshape(BN, K, 64)
    agg = jnp.sum(w * vpe, axis=1)
    out_ref[0] = _dot(agg, fw_ref[...]) + fb_ref[...] + xc


def _attn(tbl, g, wq, wkv, pw1p, pb1, pw2, pb2, aw1, ab1, aw2, ab2, fw, fb):
    b, n, _ = tbl.shape
    full = lambda s: pl.BlockSpec(s, lambda bi, i: tuple(0 for _ in s))
    return pl.pallas_call(
        _attn_body,
        grid=(b, n // BN),
        in_specs=[
            pl.BlockSpec((1, BN, TW), lambda bi, i: (bi, i, 0)),
            pl.BlockSpec((1, BN * K, TW), lambda bi, i: (bi, i, 0)),
            full((64, 64)), full((64, 128)), full((16, 64)), full((1, 64)),
            full((64, 64)), full((1, 64)), full((64, 64)), full((1, 64)),
            full((64, 64)), full((1, 64)), full((64, 64)), full((1, 64)),
        ],
        out_specs=pl.BlockSpec((1, BN, 64), lambda bi, i: (bi, i, 0)),
        out_shape=jax.ShapeDtypeStruct((b, n, 64), jnp.float32),
    )(tbl, g, wq, wkv, pw1p, pb1, pw2, pb2, aw1, ab1, aw2, ab2, fw, fb)


def kernel(x, pos, wq, wk, wv, pw1, pb1, pw2, pb2, aw1, ab1, aw2, ab2, fw, fb):
    b, n, dim = x.shape
    tot = b * n * K

    tbl = jnp.concatenate(
        [x, pos, jnp.zeros((b, n, TW - dim - 3), jnp.float32)], axis=-1)
    post = jnp.transpose(pos, (0, 2, 1))
    wkv = jnp.concatenate([wk, wv], axis=1)
    pw1p = jnp.concatenate([pw1, jnp.zeros((13, dim), jnp.float32)], axis=0)
    r1 = lambda v: v.reshape(1, -1)

    knn = _knn(pos, post)
    info = plsc.get_sparse_core_info()
    nw = info.num_cores * info.num_subcores
    idx3 = knn.reshape(nw, tot // (nw * CHUNK), CHUNK)
    g = _sc_gather(tbl.reshape(b * n, TW), idx3, tot)
    out = _attn(tbl, g.reshape(b, n * K, TW), wq, wkv, pw1p, r1(pb1), pw2,
                r1(pb2), aw1, r1(ab1), aw2, r1(ab2), fw, r1(fb))
    return out


# speedup vs baseline: 14.4259x; 1.2465x over previous
import functools

import jax
import jax.numpy as jnp
from jax import lax
from jax.experimental import pallas as pl
from jax.experimental.pallas import tpu as pltpu
from jax.experimental.pallas import tpu_sc as plsc

K = 16
TW = 128
RB = 128
BN = 128
CHUNK = 128
NBUF = 4


def _knn_body(pos_ref, post_ref, out_ref):
    b = pl.program_id(0)
    n_total = post_ref.shape[2]
    pc = pos_ref[0]
    pt = post_ref[0]
    acc = jnp.zeros((pc.shape[0], n_total), jnp.float32)
    for d in range(3):
        diff = pc[:, d:d + 1] - pt[d:d + 1, :]
        acc = acc + diff * diff
    iota = lax.broadcasted_iota(jnp.int32, acc.shape, 1)
    cols = []
    for _ in range(K):
        m = jnp.min(acc, axis=1, keepdims=True)
        hit = acc <= m
        idx = jnp.min(jnp.where(hit, iota, n_total), axis=1, keepdims=True)
        cols.append(idx)
        acc = jnp.where(hit, jnp.inf, acc)
    out_ref[0] = jnp.concatenate(cols, axis=1) + b * n_total


def _knn(pos, post):
    b, n, _ = pos.shape
    return pl.pallas_call(
        _knn_body,
        grid=(b, n // RB),
        in_specs=[
            pl.BlockSpec((1, RB, 3), lambda bi, i: (bi, i, 0)),
            pl.BlockSpec((1, 3, n), lambda bi, i: (bi, 0, 0)),
        ],
        out_specs=pl.BlockSpec((1, RB, K), lambda bi, i: (bi, i, 0)),
        out_shape=jax.ShapeDtypeStruct((b, n, K), jnp.int32),
    )(pos, post)


def _sc_gather(tbl2, idx3, tot):
    info = plsc.get_sparse_core_info()
    nc, ns = info.num_cores, info.num_subcores
    nw = nc * ns
    cpw = idx3.shape[1]
    mesh = plsc.VectorSubcoreMesh(core_axis_name="c", subcore_axis_name="s")

    @functools.partial(
        pl.kernel,
        mesh=mesh,
        out_type=jax.ShapeDtypeStruct((tot, TW), jnp.float32),
        scratch_types=(
            [pltpu.VMEM((cpw, CHUNK), jnp.int32)]
            + [pltpu.VMEM((CHUNK, TW), jnp.float32) for _ in range(NBUF)]
            + [pltpu.SemaphoreType.DMA, pltpu.SemaphoreType.DMA]
        ),
    )
    def k(tbl_hbm, idx_hbm, out_hbm, idx_v, *rest):
        bufs = rest[:NBUF]
        gsem, ssem = rest[NBUF], rest[NBUF + 1]
        wid = lax.axis_index("s") * nc + lax.axis_index("c")
        pltpu.sync_copy(idx_hbm.at[wid], idx_v)
        for r in range(cpw // NBUF):
            gets = []
            for j in range(NBUF):
                c = r * NBUF + j
                gets.append(
                    pltpu.async_copy(tbl_hbm.at[idx_v.at[c]], bufs[j], gsem))
            for g in gets:
                g.wait()
            puts = []
            for j in range(NBUF):
                c = r * NBUF + j
                base = (wid * cpw + c) * CHUNK
                puts.append(
                    pltpu.async_copy(bufs[j], out_hbm.at[pl.ds(base, CHUNK)],
                                     ssem))
            for p in puts:
                p.wait()

    return k(tbl2, idx3)


def _dot(a, b):
    return jnp.dot(a, b, precision=lax.Precision.HIGHEST,
                   preferred_element_type=jnp.float32)


def _tbl_body(x_ref, p16_ref, pw1_ref, out_ref):
    out_ref[0] = jnp.concatenate(
        [x_ref[0], _dot(p16_ref[0], pw1_ref[...])], axis=1)


def _tbl(x, p16, pw1p):
    b, n, dim = x.shape
    return pl.pallas_call(
        _tbl_body,
        grid=(b, n // 256),
        in_specs=[
            pl.BlockSpec((1, 256, dim), lambda bi, i: (bi, i, 0)),
            pl.BlockSpec((1, 256, 16), lambda bi, i: (bi, i, 0)),
            pl.BlockSpec((16, dim), lambda bi, i: (0, 0)),
        ],
        out_specs=pl.BlockSpec((1, 256, TW), lambda bi, i: (bi, i, 0)),
        out_shape=jax.ShapeDtypeStruct((b, n, TW), jnp.float32),
    )(x, p16, pw1p)


def _attn_body(tbl_ref, g_ref, wq_ref, wkv_ref, pb1_ref, pw2_ref,
               pb2_ref, aw1_ref, ab1_ref, aw2_ref, ab2_ref, fw_ref, fb_ref,
               out_ref):
    tbl = tbl_ref[0]
    g = g_ref[0]
    xc = tbl[:, 0:64]
    pic = tbl[:, 64:128]
    xg = g[:, 0:64]
    pjg = g[:, 64:128]

    q = _dot(xc, wq_ref[...])
    kv = _dot(xg, wkv_ref[...])
    pre = pic[:, None, :] - pjg.reshape(BN, K, 64) + pb1_ref[...]
    pe = _dot(jax.nn.relu(pre.reshape(BN * K, 64)),
              pw2_ref[...]) + pb2_ref[...]
    h = (q[:, None, :] - kv[:, 0:64].reshape(BN, K, 64)
         + pe.reshape(BN, K, 64))
    a = _dot(jax.nn.relu(_dot(h.reshape(BN * K, 64), aw1_ref[...])
                         + ab1_ref[...]), aw2_ref[...]) + ab2_ref[...]
    s = a.reshape(BN, K, 64) * 0.125
    s = s - jnp.max(s, axis=1, keepdims=True)
    e = jnp.exp(s)
    w = e / jnp.sum(e, axis=1, keepdims=True)
    vpe = kv[:, 64:128].reshape(BN, K, 64) + pe.reshape(BN, K, 64)
    agg = jnp.sum(w * vpe, axis=1)
    out_ref[0] = _dot(agg, fw_ref[...]) + fb_ref[...] + xc


def _attn(tbl, g, wq, wkv, pb1, pw2, pb2, aw1, ab1, aw2, ab2, fw, fb):
    b, n, _ = tbl.shape
    full = lambda s: pl.BlockSpec(s, lambda bi, i: tuple(0 for _ in s))
    return pl.pallas_call(
        _attn_body,
        grid=(b, n // BN),
        in_specs=[
            pl.BlockSpec((1, BN, TW), lambda bi, i: (bi, i, 0)),
            pl.BlockSpec((1, BN * K, TW), lambda bi, i: (bi, i, 0)),
            full((64, 64)), full((64, 128)), full((1, 64)),
            full((64, 64)), full((1, 64)), full((64, 64)), full((1, 64)),
            full((64, 64)), full((1, 64)), full((64, 64)), full((1, 64)),
        ],
        out_specs=pl.BlockSpec((1, BN, 64), lambda bi, i: (bi, i, 0)),
        out_shape=jax.ShapeDtypeStruct((b, n, 64), jnp.float32),
    )(tbl, g, wq, wkv, pb1, pw2, pb2, aw1, ab1, aw2, ab2, fw, fb)


def kernel(x, pos, wq, wk, wv, pw1, pb1, pw2, pb2, aw1, ab1, aw2, ab2, fw, fb):
    b, n, dim = x.shape
    tot = b * n * K

    p16 = jnp.concatenate([pos, jnp.zeros((b, n, 13), jnp.float32)], axis=-1)
    post = jnp.transpose(pos, (0, 2, 1))
    wkv = jnp.concatenate([wk, wv], axis=1)
    pw1p = jnp.concatenate([pw1, jnp.zeros((13, dim), jnp.float32)], axis=0)
    r1 = lambda v: v.reshape(1, -1)

    tbl = _tbl(x, p16, pw1p)
    knn = _knn(pos, post)
    info = plsc.get_sparse_core_info()
    nw = info.num_cores * info.num_subcores
    idx3 = knn.reshape(nw, tot // (nw * CHUNK), CHUNK)
    g = _sc_gather(tbl.reshape(b * n, TW), idx3, tot)
    out = _attn(tbl, g.reshape(b, n * K, TW), wq, wkv, r1(pb1), pw2,
                r1(pb2), aw1, r1(ab1), aw2, r1(ab2), fw, r1(fb))
    return out


# speedup vs baseline: 20.1195x; 1.3947x over previous
import functools

import jax
import jax.numpy as jnp
from jax import lax
from jax.experimental import pallas as pl
from jax.experimental.pallas import tpu as pltpu
from jax.experimental.pallas import tpu_sc as plsc

K = 16
TW = 128
RB = 128
BN = 128
CHUNK = 128
NBUF = 4


def _knn_body(pos_ref, post_ref, out_ref):
    b = pl.program_id(0)
    n_total = post_ref.shape[2]
    pc = pos_ref[0]
    pt = post_ref[0]
    acc = jnp.zeros((pc.shape[0], n_total), jnp.float32)
    for d in range(3):
        diff = pc[:, d:d + 1] - pt[d:d + 1, :]
        acc = acc + diff * diff
    iota = lax.broadcasted_iota(jnp.int32, acc.shape, 1)
    cols = []
    for _ in range(K):
        m = jnp.min(acc, axis=1, keepdims=True)
        hit = acc <= m
        idx = jnp.min(jnp.where(hit, iota, n_total), axis=1, keepdims=True)
        cols.append(idx)
        acc = jnp.where(hit, jnp.inf, acc)
    out_ref[0] = jnp.concatenate(cols, axis=1) + b * n_total


def _knn(pos, post):
    b, n, _ = pos.shape
    return pl.pallas_call(
        _knn_body,
        grid=(b, n // RB),
        in_specs=[
            pl.BlockSpec((1, RB, 3), lambda bi, i: (bi, i, 0)),
            pl.BlockSpec((1, 3, n), lambda bi, i: (bi, 0, 0)),
        ],
        out_specs=pl.BlockSpec((1, RB, K), lambda bi, i: (bi, i, 0)),
        out_shape=jax.ShapeDtypeStruct((b, n, K), jnp.int32),
    )(pos, post)


def _sc_gather(tbl2, idx3, tot):
    info = plsc.get_sparse_core_info()
    nc, ns = info.num_cores, info.num_subcores
    nw = nc * ns
    cpw = idx3.shape[1]
    mesh = plsc.VectorSubcoreMesh(core_axis_name="c", subcore_axis_name="s")

    @functools.partial(
        pl.kernel,
        mesh=mesh,
        out_type=jax.ShapeDtypeStruct((tot, TW), jnp.float32),
        scratch_types=(
            [pltpu.VMEM((cpw, CHUNK), jnp.int32)]
            + [pltpu.VMEM((CHUNK, TW), jnp.float32) for _ in range(NBUF)]
            + [pltpu.SemaphoreType.DMA, pltpu.SemaphoreType.DMA]
        ),
    )
    def k(tbl_hbm, idx_hbm, out_hbm, idx_v, *rest):
        bufs = rest[:NBUF]
        gsem, ssem = rest[NBUF], rest[NBUF + 1]
        wid = lax.axis_index("s") * nc + lax.axis_index("c")
        pltpu.sync_copy(idx_hbm.at[wid], idx_v)
        for r in range(cpw // NBUF):
            gets = []
            for j in range(NBUF):
                c = r * NBUF + j
                gets.append(
                    pltpu.async_copy(tbl_hbm.at[idx_v.at[c]], bufs[j], gsem))
            for g in gets:
                g.wait()
            puts = []
            for j in range(NBUF):
                c = r * NBUF + j
                base = (wid * cpw + c) * CHUNK
                puts.append(
                    pltpu.async_copy(bufs[j], out_hbm.at[pl.ds(base, CHUNK)],
                                     ssem))
            for p in puts:
                p.wait()

    return k(tbl2, idx3)


def _dot(a, b):
    return jnp.dot(a, b, precision=lax.Precision.DEFAULT,
                   preferred_element_type=jnp.float32)


def _tbl_body(x_ref, p16_ref, pw1_ref, out_ref):
    out_ref[0] = jnp.concatenate(
        [x_ref[0], _dot(p16_ref[0], pw1_ref[...])], axis=1)


def _tbl(x, p16, pw1p):
    b, n, dim = x.shape
    return pl.pallas_call(
        _tbl_body,
        grid=(b, n // 256),
        in_specs=[
            pl.BlockSpec((1, 256, dim), lambda bi, i: (bi, i, 0)),
            pl.BlockSpec((1, 256, 16), lambda bi, i: (bi, i, 0)),
            pl.BlockSpec((16, dim), lambda bi, i: (0, 0)),
        ],
        out_specs=pl.BlockSpec((1, 256, TW), lambda bi, i: (bi, i, 0)),
        out_shape=jax.ShapeDtypeStruct((b, n, TW), jnp.float32),
    )(x, p16, pw1p)


def _attn_body(tbl_ref, g_ref, wq_ref, wkv_ref, pb1_ref, pw2_ref,
               pb2_ref, aw1_ref, ab1_ref, aw2_ref, ab2_ref, fw_ref, fb_ref,
               out_ref):
    tbl = tbl_ref[0]
    g = g_ref[0]
    xc = tbl[:, 0:64]
    pic = tbl[:, 64:128]
    xg = g[:, 0:64]
    pjg = g[:, 64:128]

    q = _dot(xc, wq_ref[...])
    kv = _dot(xg, wkv_ref[...])
    pre = pic[:, None, :] - pjg.reshape(BN, K, 64) + pb1_ref[...]
    pe = _dot(jax.nn.relu(pre.reshape(BN * K, 64)),
              pw2_ref[...]) + pb2_ref[...]
    h = (q[:, None, :] - kv[:, 0:64].reshape(BN, K, 64)
         + pe.reshape(BN, K, 64))
    a = _dot(jax.nn.relu(_dot(h.reshape(BN * K, 64), aw1_ref[...])
                         + ab1_ref[...]), aw2_ref[...]) + ab2_ref[...]
    s = a.reshape(BN, K, 64) * 0.125
    s = s - jnp.max(s, axis=1, keepdims=True)
    e = jnp.exp(s)
    w = e / jnp.sum(e, axis=1, keepdims=True)
    vpe = kv[:, 64:128].reshape(BN, K, 64) + pe.reshape(BN, K, 64)
    agg = jnp.sum(w * vpe, axis=1)
    out_ref[0] = _dot(agg, fw_ref[...]) + fb_ref[...] + xc


def _attn(tbl, g, wq, wkv, pb1, pw2, pb2, aw1, ab1, aw2, ab2, fw, fb):
    b, n, _ = tbl.shape
    full = lambda s: pl.BlockSpec(s, lambda bi, i: tuple(0 for _ in s))
    return pl.pallas_call(
        _attn_body,
        grid=(b, n // BN),
        in_specs=[
            pl.BlockSpec((1, BN, TW), lambda bi, i: (bi, i, 0)),
            pl.BlockSpec((1, BN * K, TW), lambda bi, i: (bi, i, 0)),
            full((64, 64)), full((64, 128)), full((1, 64)),
            full((64, 64)), full((1, 64)), full((64, 64)), full((1, 64)),
            full((64, 64)), full((1, 64)), full((64, 64)), full((1, 64)),
        ],
        out_specs=pl.BlockSpec((1, BN, 64), lambda bi, i: (bi, i, 0)),
        out_shape=jax.ShapeDtypeStruct((b, n, 64), jnp.float32),
    )(tbl, g, wq, wkv, pb1, pw2, pb2, aw1, ab1, aw2, ab2, fw, fb)


def kernel(x, pos, wq, wk, wv, pw1, pb1, pw2, pb2, aw1, ab1, aw2, ab2, fw, fb):
    b, n, dim = x.shape
    tot = b * n * K

    p16 = jnp.concatenate([pos, jnp.zeros((b, n, 13), jnp.float32)], axis=-1)
    post = jnp.transpose(pos, (0, 2, 1))
    wkv = jnp.concatenate([wk, wv], axis=1)
    pw1p = jnp.concatenate([pw1, jnp.zeros((13, dim), jnp.float32)], axis=0)
    r1 = lambda v: v.reshape(1, -1)

    tbl = _tbl(x, p16, pw1p)
    knn = _knn(pos, post)
    info = plsc.get_sparse_core_info()
    nw = info.num_cores * info.num_subcores
    idx3 = knn.reshape(nw, tot // (nw * CHUNK), CHUNK)
    g = _sc_gather(tbl.reshape(b * n, TW), idx3, tot)
    out = _attn(tbl, g.reshape(b, n * K, TW), wq, wkv, r1(pb1), pw2,
                r1(pb2), aw1, r1(ab1), aw2, r1(ab2), fw, r1(fb))
    return out


# speedup vs baseline: 23.6203x; 1.1740x over previous
import functools

import jax
import jax.numpy as jnp
from jax import lax
from jax.experimental import pallas as pl
from jax.experimental.pallas import tpu as pltpu
from jax.experimental.pallas import tpu_sc as plsc

K = 16
TW = 128
RB = 128
BN = 128
CHUNK = 128
NBUF = 4


SEG = 256
DPT = 3


def _topk_cols(vals, idxs, n_total):
    cols = []
    last = None
    for _ in range(K):
        m = jnp.min(vals, axis=1, keepdims=True)
        hit = vals <= m
        cols.append(jnp.min(jnp.where(hit, idxs, n_total), axis=1,
                            keepdims=True))
        vals = jnp.where(hit, jnp.inf, vals)
        last = m
    return jnp.concatenate(cols, axis=1), last


def _dists(pos_ref, post_ref):
    n_total = post_ref.shape[2]
    pc = pos_ref[0]
    pt = post_ref[0]
    acc = jnp.zeros((pc.shape[0], n_total), jnp.float32)
    for d in range(3):
        diff = pc[:, d:d + 1] - pt[d:d + 1, :]
        acc = acc + diff * diff
    return acc


def _knn_body(pos_ref, post_ref, out_ref):
    b = pl.program_id(0)
    n_total = post_ref.shape[2]
    acc = _dists(pos_ref, post_ref)
    iota = lax.broadcasted_iota(jnp.int32, acc.shape, 1)
    rows = acc.shape[0]
    a3 = acc.reshape(rows, n_total // SEG, SEG)
    i3 = iota.reshape(rows, n_total // SEG, SEG)
    cvals, cidxs = [], []
    cur = a3
    for _ in range(DPT):
        m = jnp.min(cur, axis=1)
        hit = cur <= m[:, None, :]
        mi = jnp.min(jnp.where(hit, i3, n_total), axis=1)
        cvals.append(m)
        cidxs.append(mi)
        cur = jnp.where(i3 == mi[:, None, :], jnp.inf, cur)
    m_next = jnp.min(cur, axis=1)
    cand = jnp.concatenate(cvals, axis=1)
    cidx = jnp.concatenate(cidxs, axis=1)
    cols, last = _topk_cols(cand, cidx, n_total)
    out_ref[0] = cols + b * n_total
    bad = jnp.max(jnp.where(m_next <= last, 1, 0))

    @pl.when(bad > 0)
    def _():
        acc2 = _dists(pos_ref, post_ref)
        cols2, _ = _topk_cols(acc2,
                              lax.broadcasted_iota(jnp.int32, acc2.shape, 1),
                              n_total)
        out_ref[0] = cols2 + b * n_total


def _knn(pos, post):
    b, n, _ = pos.shape
    return pl.pallas_call(
        _knn_body,
        grid=(b, n // RB),
        in_specs=[
            pl.BlockSpec((1, RB, 3), lambda bi, i: (bi, i, 0)),
            pl.BlockSpec((1, 3, n), lambda bi, i: (bi, 0, 0)),
        ],
        out_specs=pl.BlockSpec((1, RB, K), lambda bi, i: (bi, i, 0)),
        out_shape=jax.ShapeDtypeStruct((b, n, K), jnp.int32),
    )(pos, post)


def _sc_gather(tbl2, idx3, tot):
    info = plsc.get_sparse_core_info()
    nc, ns = info.num_cores, info.num_subcores
    nw = nc * ns
    cpw = idx3.shape[1]
    mesh = plsc.VectorSubcoreMesh(core_axis_name="c", subcore_axis_name="s")

    @functools.partial(
        pl.kernel,
        mesh=mesh,
        out_type=jax.ShapeDtypeStruct((tot, TW), jnp.float32),
        scratch_types=(
            [pltpu.VMEM((cpw, CHUNK), jnp.int32)]
            + [pltpu.VMEM((CHUNK, TW), jnp.float32) for _ in range(NBUF)]
            + [pltpu.SemaphoreType.DMA, pltpu.SemaphoreType.DMA]
        ),
    )
    def k(tbl_hbm, idx_hbm, out_hbm, idx_v, *rest):
        bufs = rest[:NBUF]
        gsem, ssem = rest[NBUF], rest[NBUF + 1]
        wid = lax.axis_index("s") * nc + lax.axis_index("c")
        pltpu.sync_copy(idx_hbm.at[wid], idx_v)
        for r in range(cpw // NBUF):
            gets = []
            for j in range(NBUF):
                c = r * NBUF + j
                gets.append(
                    pltpu.async_copy(tbl_hbm.at[idx_v.at[c]], bufs[j], gsem))
            for g in gets:
                g.wait()
            puts = []
            for j in range(NBUF):
                c = r * NBUF + j
                base = (wid * cpw + c) * CHUNK
                puts.append(
                    pltpu.async_copy(bufs[j], out_hbm.at[pl.ds(base, CHUNK)],
                                     ssem))
            for p in puts:
                p.wait()

    return k(tbl2, idx3)


def _dot(a, b):
    return jnp.dot(a, b, precision=lax.Precision.DEFAULT,
                   preferred_element_type=jnp.float32)


def _tbl_body(x_ref, p16_ref, pw1_ref, out_ref):
    out_ref[0] = jnp.concatenate(
        [x_ref[0], _dot(p16_ref[0], pw1_ref[...])], axis=1)


def _tbl(x, p16, pw1p):
    b, n, dim = x.shape
    return pl.pallas_call(
        _tbl_body,
        grid=(b, n // 256),
        in_specs=[
            pl.BlockSpec((1, 256, dim), lambda bi, i: (bi, i, 0)),
            pl.BlockSpec((1, 256, 16), lambda bi, i: (bi, i, 0)),
            pl.BlockSpec((16, dim), lambda bi, i: (0, 0)),
        ],
        out_specs=pl.BlockSpec((1, 256, TW), lambda bi, i: (bi, i, 0)),
        out_shape=jax.ShapeDtypeStruct((b, n, TW), jnp.float32),
    )(x, p16, pw1p)


def _attn_body(tbl_ref, g_ref, wq_ref, wkv_ref, pb1_ref, pw2_ref,
               pb2_ref, aw1_ref, ab1_ref, aw2_ref, ab2_ref, fw_ref, fb_ref,
               out_ref):
    tbl = tbl_ref[0]
    g = g_ref[0]
    xc = tbl[:, 0:64]
    pic = tbl[:, 64:128]
    xg = g[:, 0:64]
    pjg = g[:, 64:128]

    q = _dot(xc, wq_ref[...])
    kv = _dot(xg, wkv_ref[...])
    pre = pic[:, None, :] - pjg.reshape(BN, K, 64) + pb1_ref[...]
    pe = _dot(jax.nn.relu(pre.reshape(BN * K, 64)),
              pw2_ref[...]) + pb2_ref[...]
    h = (q[:, None, :] - kv[:, 0:64].reshape(BN, K, 64)
         + pe.reshape(BN, K, 64))
    a = _dot(jax.nn.relu(_dot(h.reshape(BN * K, 64), aw1_ref[...])
                         + ab1_ref[...]), aw2_ref[...]) + ab2_ref[...]
    s = a.reshape(BN, K, 64) * 0.125
    s = s - jnp.max(s, axis=1, keepdims=True)
    e = jnp.exp(s)
    w = e / jnp.sum(e, axis=1, keepdims=True)
    vpe = kv[:, 64:128].reshape(BN, K, 64) + pe.reshape(BN, K, 64)
    agg = jnp.sum(w * vpe, axis=1)
    out_ref[0] = _dot(agg, fw_ref[...]) + fb_ref[...] + xc


def _attn(tbl, g, wq, wkv, pb1, pw2, pb2, aw1, ab1, aw2, ab2, fw, fb):
    b, n, _ = tbl.shape
    full = lambda s: pl.BlockSpec(s, lambda bi, i: tuple(0 for _ in s))
    return pl.pallas_call(
        _attn_body,
        grid=(b, n // BN),
        in_specs=[
            pl.BlockSpec((1, BN, TW), lambda bi, i: (bi, i, 0)),
            pl.BlockSpec((1, BN * K, TW), lambda bi, i: (bi, i, 0)),
            full((64, 64)), full((64, 128)), full((1, 64)),
            full((64, 64)), full((1, 64)), full((64, 64)), full((1, 64)),
            full((64, 64)), full((1, 64)), full((64, 64)), full((1, 64)),
        ],
        out_specs=pl.BlockSpec((1, BN, 64), lambda bi, i: (bi, i, 0)),
        out_shape=jax.ShapeDtypeStruct((b, n, 64), jnp.float32),
    )(tbl, g, wq, wkv, pb1, pw2, pb2, aw1, ab1, aw2, ab2, fw, fb)


def kernel(x, pos, wq, wk, wv, pw1, pb1, pw2, pb2, aw1, ab1, aw2, ab2, fw, fb):
    b, n, dim = x.shape
    tot = b * n * K

    p16 = jnp.concatenate([pos, jnp.zeros((b, n, 13), jnp.float32)], axis=-1)
    post = jnp.transpose(pos, (0, 2, 1))
    wkv = jnp.concatenate([wk, wv], axis=1)
    pw1p = jnp.concatenate([pw1, jnp.zeros((13, dim), jnp.float32)], axis=0)
    r1 = lambda v: v.reshape(1, -1)

    tbl = _tbl(x, p16, pw1p)
    knn = _knn(pos, post)
    info = plsc.get_sparse_core_info()
    nw = info.num_cores * info.num_subcores
    idx3 = knn.reshape(nw, tot // (nw * CHUNK), CHUNK)
    g = _sc_gather(tbl.reshape(b * n, TW), idx3, tot)
    out = _attn(tbl, g.reshape(b, n * K, TW), wq, wkv, r1(pb1), pw2,
                r1(pb2), aw1, r1(ab1), aw2, r1(ab2), fw, r1(fb))
    return out


# speedup vs baseline: 24.2343x; 1.0260x over previous
import functools

import jax
import jax.numpy as jnp
from jax import lax
from jax.experimental import pallas as pl
from jax.experimental.pallas import tpu as pltpu
from jax.experimental.pallas import tpu_sc as plsc

K = 16
TW = 128
RB = 128
BN = 128
CHUNK = 128
NBUF = 4


SEG = 256
DPT = 3


def _topk_cols(vals, idxs, n_total):
    cols = []
    last = None
    for _ in range(K):
        m = jnp.min(vals, axis=1, keepdims=True)
        hit = vals <= m
        cols.append(jnp.min(jnp.where(hit, idxs, n_total), axis=1,
                            keepdims=True))
        vals = jnp.where(hit, jnp.inf, vals)
        last = m
    return jnp.concatenate(cols, axis=1), last


def _dists(pos_ref, post_ref):
    n_total = post_ref.shape[2]
    pc = pos_ref[0]
    pt = post_ref[0]
    acc = jnp.zeros((pc.shape[0], n_total), jnp.float32)
    for d in range(3):
        diff = pc[:, d:d + 1] - pt[d:d + 1, :]
        acc = acc + diff * diff
    return acc


def _knn_body(pos_ref, post_ref, x_ref, pw1_ref, out_ref, tbl_ref):
    b = pl.program_id(0)
    n_total = post_ref.shape[2]
    tbl_ref[0] = jnp.concatenate(
        [x_ref[0], _dot(pos_ref[0], pw1_ref[...])], axis=1)
    acc = _dists(pos_ref, post_ref)
    iota = lax.broadcasted_iota(jnp.int32, acc.shape, 1)
    rows = acc.shape[0]
    a3 = acc.reshape(rows, n_total // SEG, SEG)
    i3 = iota.reshape(rows, n_total // SEG, SEG)
    cvals, cidxs = [], []
    cur = a3
    for _ in range(DPT):
        m = jnp.min(cur, axis=1)
        hit = cur <= m[:, None, :]
        mi = jnp.min(jnp.where(hit, i3, n_total), axis=1)
        cvals.append(m)
        cidxs.append(mi)
        cur = jnp.where(i3 == mi[:, None, :], jnp.inf, cur)
    m_next = jnp.min(cur, axis=1)
    cand = jnp.concatenate(cvals, axis=1)
    cidx = jnp.concatenate(cidxs, axis=1)
    cols, last = _topk_cols(cand, cidx, n_total)
    out_ref[0] = cols + b * n_total
    bad = jnp.max(jnp.where(m_next <= last, 1, 0))

    @pl.when(bad > 0)
    def _():
        acc2 = _dists(pos_ref, post_ref)
        cols2, _ = _topk_cols(acc2,
                              lax.broadcasted_iota(jnp.int32, acc2.shape, 1),
                              n_total)
        out_ref[0] = cols2 + b * n_total


def _knn(pos, post, x, pw1):
    b, n, dim = x.shape
    return pl.pallas_call(
        _knn_body,
        grid=(b, n // RB),
        in_specs=[
            pl.BlockSpec((1, RB, 3), lambda bi, i: (bi, i, 0)),
            pl.BlockSpec((1, 3, n), lambda bi, i: (bi, 0, 0)),
            pl.BlockSpec((1, RB, dim), lambda bi, i: (bi, i, 0)),
            pl.BlockSpec((3, dim), lambda bi, i: (0, 0)),
        ],
        out_specs=(
            pl.BlockSpec((1, RB, K), lambda bi, i: (bi, i, 0)),
            pl.BlockSpec((1, RB, TW), lambda bi, i: (bi, i, 0)),
        ),
        out_shape=(
            jax.ShapeDtypeStruct((b, n, K), jnp.int32),
            jax.ShapeDtypeStruct((b, n, TW), jnp.float32),
        ),
    )(pos, post, x, pw1)


def _sc_gather(tbl2, idx3, tot):
    info = plsc.get_sparse_core_info()
    nc, ns = info.num_cores, info.num_subcores
    nw = nc * ns
    cpw = idx3.shape[1]
    mesh = plsc.VectorSubcoreMesh(core_axis_name="c", subcore_axis_name="s")

    @functools.partial(
        pl.kernel,
        mesh=mesh,
        out_type=jax.ShapeDtypeStruct((tot, TW), jnp.float32),
        scratch_types=(
            [pltpu.VMEM((cpw, CHUNK), jnp.int32)]
            + [pltpu.VMEM((CHUNK, TW), jnp.float32) for _ in range(NBUF)]
            + [pltpu.SemaphoreType.DMA, pltpu.SemaphoreType.DMA]
        ),
    )
    def k(tbl_hbm, idx_hbm, out_hbm, idx_v, *rest):
        bufs = rest[:NBUF]
        gsem, ssem = rest[NBUF], rest[NBUF + 1]
        wid = lax.axis_index("s") * nc + lax.axis_index("c")
        pltpu.sync_copy(idx_hbm.at[wid], idx_v)
        for r in range(cpw // NBUF):
            gets = []
            for j in range(NBUF):
                c = r * NBUF + j
                gets.append(
                    pltpu.async_copy(tbl_hbm.at[idx_v.at[c]], bufs[j], gsem))
            for g in gets:
                g.wait()
            puts = []
            for j in range(NBUF):
                c = r * NBUF + j
                base = (wid * cpw + c) * CHUNK
                puts.append(
                    pltpu.async_copy(bufs[j], out_hbm.at[pl.ds(base, CHUNK)],
                                     ssem))
            for p in puts:
                p.wait()

    return k(tbl2, idx3)


def _dot(a, b):
    return jnp.dot(a, b, precision=lax.Precision.DEFAULT,
                   preferred_element_type=jnp.float32)


def _attn_body(tbl_ref, g_ref, wq_ref, wkv_ref, pb1_ref, pw2_ref,
               pb2_ref, aw1_ref, ab1_ref, aw2_ref, ab2_ref, fw_ref, fb_ref,
               out_ref):
    tbl = tbl_ref[0]
    g = g_ref[0]
    xc = tbl[:, 0:64]
    pic = tbl[:, 64:128]
    xg = g[:, 0:64]
    pjg = g[:, 64:128]

    q = _dot(xc, wq_ref[...])
    kv = _dot(xg, wkv_ref[...])
    pre = pic[:, None, :] - pjg.reshape(BN, K, 64) + pb1_ref[...]
    pe = _dot(jax.nn.relu(pre.reshape(BN * K, 64)),
              pw2_ref[...]) + pb2_ref[...]
    h = (q[:, None, :] - kv[:, 0:64].reshape(BN, K, 64)
         + pe.reshape(BN, K, 64))
    a = _dot(jax.nn.relu(_dot(h.reshape(BN * K, 64), aw1_ref[...])
                         + ab1_ref[...]), aw2_ref[...]) + ab2_ref[...]
    s = a.reshape(BN, K, 64) * 0.125
    s = s - jnp.max(s, axis=1, keepdims=True)
    e = jnp.exp(s)
    w = e / jnp.sum(e, axis=1, keepdims=True)
    vpe = kv[:, 64:128].reshape(BN, K, 64) + pe.reshape(BN, K, 64)
    agg = jnp.sum(w * vpe, axis=1)
    out_ref[0] = _dot(agg, fw_ref[...]) + fb_ref[...] + xc


def _attn(tbl, g, wq, wkv, pb1, pw2, pb2, aw1, ab1, aw2, ab2, fw, fb):
    b, n, _ = tbl.shape
    full = lambda s: pl.BlockSpec(s, lambda bi, i: tuple(0 for _ in s))
    return pl.pallas_call(
        _attn_body,
        grid=(b, n // BN),
        in_specs=[
            pl.BlockSpec((1, BN, TW), lambda bi, i: (bi, i, 0)),
            pl.BlockSpec((1, BN * K, TW), lambda bi, i: (bi, i, 0)),
            full((64, 64)), full((64, 128)), full((1, 64)),
            full((64, 64)), full((1, 64)), full((64, 64)), full((1, 64)),
            full((64, 64)), full((1, 64)), full((64, 64)), full((1, 64)),
        ],
        out_specs=pl.BlockSpec((1, BN, 64), lambda bi, i: (bi, i, 0)),
        out_shape=jax.ShapeDtypeStruct((b, n, 64), jnp.float32),
    )(tbl, g, wq, wkv, pb1, pw2, pb2, aw1, ab1, aw2, ab2, fw, fb)


def kernel(x, pos, wq, wk, wv, pw1, pb1, pw2, pb2, aw1, ab1, aw2, ab2, fw, fb):
    b, n, dim = x.shape
    tot = b * n * K

    post = jnp.transpose(pos, (0, 2, 1))
    wkv = jnp.concatenate([wk, wv], axis=1)
    r1 = lambda v: v.reshape(1, -1)

    knn, tbl = _knn(pos, post, x, pw1)
    info = plsc.get_sparse_core_info()
    nw = info.num_cores * info.num_subcores
    idx3 = knn.reshape(nw, tot // (nw * CHUNK), CHUNK)
    g = _sc_gather(tbl.reshape(b * n, TW), idx3, tot)
    out = _attn(tbl, g.reshape(b, n * K, TW), wq, wkv, r1(pb1), pw2,
                r1(pb2), aw1, r1(ab1), aw2, r1(ab2), fw, r1(fb))
    return out


# speedup vs baseline: 24.8373x; 1.0249x over previous
import functools

import jax
import jax.numpy as jnp
from jax import lax
from jax.experimental import pallas as pl
from jax.experimental.pallas import tpu as pltpu
from jax.experimental.pallas import tpu_sc as plsc

K = 16
TW = 128
RB = 128
BN = 128
CHUNK = 128
NBUF = 4


SEG = 256
DPT = 3


def _topk_cols(vals, idxs, n_total):
    cols = []
    last = None
    for _ in range(K):
        m = jnp.min(vals, axis=1, keepdims=True)
        hit = vals <= m
        cols.append(jnp.min(jnp.where(hit, idxs, n_total), axis=1,
                            keepdims=True))
        vals = jnp.where(hit, jnp.inf, vals)
        last = m
    return jnp.concatenate(cols, axis=1), last


def _dists(pos_ref, post_ref):
    n_total = post_ref.shape[2]
    pc = pos_ref[0]
    pt = post_ref[0]
    acc = jnp.zeros((pc.shape[0], n_total), jnp.float32)
    for d in range(3):
        diff = pc[:, d:d + 1] - pt[d:d + 1, :]
        acc = acc + diff * diff
    return acc


def _knn_body(pos_ref, post_ref, x_ref, pw1_ref, out_ref, tbl_ref):
    b = pl.program_id(0)
    n_total = post_ref.shape[2]
    tbl_ref[0] = jnp.concatenate(
        [x_ref[0], _dot(pos_ref[0], pw1_ref[...])], axis=1)
    acc = _dists(pos_ref, post_ref)
    iota = lax.broadcasted_iota(jnp.int32, acc.shape, 1)
    rows = acc.shape[0]
    a3 = acc.reshape(rows, n_total // SEG, SEG)
    i3 = iota.reshape(rows, n_total // SEG, SEG)
    cvals, cidxs = [], []
    cur = a3
    for _ in range(DPT):
        m = jnp.min(cur, axis=1)
        hit = cur <= m[:, None, :]
        mi = jnp.min(jnp.where(hit, i3, n_total), axis=1)
        cvals.append(m)
        cidxs.append(mi)
        cur = jnp.where(i3 == mi[:, None, :], jnp.inf, cur)
    m_next = jnp.min(cur, axis=1)
    cand = jnp.concatenate(cvals, axis=1)
    cidx = jnp.concatenate(cidxs, axis=1)
    cols, last = _topk_cols(cand, cidx, n_total)
    out_ref[0] = cols + b * n_total
    bad = jnp.max(jnp.where(m_next <= last, 1, 0))

    @pl.when(bad > 0)
    def _():
        acc2 = _dists(pos_ref, post_ref)
        cols2, _ = _topk_cols(acc2,
                              lax.broadcasted_iota(jnp.int32, acc2.shape, 1),
                              n_total)
        out_ref[0] = cols2 + b * n_total


def _knn(pos, post, x, pw1):
    b, n, dim = x.shape
    return pl.pallas_call(
        _knn_body,
        grid=(b, n // RB),
        in_specs=[
            pl.BlockSpec((1, RB, 3), lambda bi, i: (bi, i, 0)),
            pl.BlockSpec((1, 3, n), lambda bi, i: (bi, 0, 0)),
            pl.BlockSpec((1, RB, dim), lambda bi, i: (bi, i, 0)),
            pl.BlockSpec((3, dim), lambda bi, i: (0, 0)),
        ],
        out_specs=(
            pl.BlockSpec((1, RB, K), lambda bi, i: (bi, i, 0)),
            pl.BlockSpec((1, RB, TW), lambda bi, i: (bi, i, 0)),
        ),
        out_shape=(
            jax.ShapeDtypeStruct((b, n, K), jnp.int32),
            jax.ShapeDtypeStruct((b, n, TW), jnp.float32),
        ),
    )(pos, post, x, pw1)


def _sc_gather(tbl2, idx3, tot):
    info = plsc.get_sparse_core_info()
    nc, ns = info.num_cores, info.num_subcores
    nw = nc * ns
    cpw = idx3.shape[1]
    mesh = plsc.VectorSubcoreMesh(core_axis_name="c", subcore_axis_name="s")

    @functools.partial(
        pl.kernel,
        mesh=mesh,
        out_type=jax.ShapeDtypeStruct((tot, TW), jnp.float32),
        scratch_types=(
            [pltpu.VMEM((cpw, CHUNK), jnp.int32)]
            + [pltpu.VMEM((CHUNK, TW), jnp.float32) for _ in range(NBUF)]
            + [pltpu.SemaphoreType.DMA, pltpu.SemaphoreType.DMA]
        ),
    )
    def k(tbl_hbm, idx_hbm, out_hbm, idx_v, *rest):
        bufs = rest[:NBUF]
        gsem, ssem = rest[NBUF], rest[NBUF + 1]
        wid = lax.axis_index("s") * nc + lax.axis_index("c")
        pltpu.sync_copy(idx_hbm.at[wid], idx_v)
        for r in range(cpw // NBUF):
            gets = []
            for j in range(NBUF):
                c = r * NBUF + j
                gets.append(
                    pltpu.async_copy(tbl_hbm.at[idx_v.at[c]], bufs[j], gsem))
            for g in gets:
                g.wait()
            puts = []
            for j in range(NBUF):
                c = r * NBUF + j
                base = (wid * cpw + c) * CHUNK
                puts.append(
                    pltpu.async_copy(bufs[j], out_hbm.at[pl.ds(base, CHUNK)],
                                     ssem))
            for p in puts:
                p.wait()

    return k(tbl2, idx3)


def _dot(a, b):
    return jnp.dot(a, b, precision=lax.Precision.DEFAULT,
                   preferred_element_type=jnp.float32)


def _attn_body(tbl_ref, g_ref, wq_ref, wkv_ref, pb1_ref, pw2_ref,
               pb2_ref, aw1_ref, ab1_ref, aw2_ref, ab2_ref, fw_ref, fb_ref,
               out_ref):
    tbl = tbl_ref[0]
    g = g_ref[0]
    xc = tbl[:, 0:64]
    pic = tbl[:, 64:128]
    xg = g[:, 0:64]
    pjg = g[:, 64:128]

    q = _dot(xc, wq_ref[...])
    kv = _dot(xg, wkv_ref[...])
    pre = pic[:, None, :] - pjg.reshape(BN, K, 64) + pb1_ref[...]
    pe = _dot(jax.nn.relu(pre.reshape(BN * K, 64)),
              pw2_ref[...]) + pb2_ref[...]
    h = (q[:, None, :] - kv[:, 0:64].reshape(BN, K, 64)
         + pe.reshape(BN, K, 64))
    a = _dot(jax.nn.relu(_dot(h.reshape(BN * K, 64), aw1_ref[...])
                         + ab1_ref[...]), aw2_ref[...]) + ab2_ref[...]
    s = a.reshape(BN, K, 64) * 0.125
    s = s - jnp.max(s, axis=1, keepdims=True)
    e = jnp.exp(s)
    w = e / jnp.sum(e, axis=1, keepdims=True)
    vpe = kv[:, 64:128].reshape(BN, K, 64) + pe.reshape(BN, K, 64)
    agg = jnp.sum(w * vpe, axis=1)
    out_ref[0] = _dot(agg, fw_ref[...]) + fb_ref[...] + xc


def _attn(tbl, g, wq, wkv, pb1, pw2, pb2, aw1, ab1, aw2, ab2, fw, fb):
    b, n, _ = tbl.shape
    full = lambda s: pl.BlockSpec(s, lambda bi, i: tuple(0 for _ in s))
    return pl.pallas_call(
        _attn_body,
        grid=(b, n // BN),
        in_specs=[
            pl.BlockSpec((1, BN, TW), lambda bi, i: (bi, i, 0)),
            pl.BlockSpec((1, BN * K, TW), lambda bi, i: (bi, i, 0)),
            full((64, 64)), full((64, 128)), full((1, 64)),
            full((64, 64)), full((1, 64)), full((64, 64)), full((1, 64)),
            full((64, 64)), full((1, 64)), full((64, 64)), full((1, 64)),
        ],
        out_specs=pl.BlockSpec((1, BN, 64), lambda bi, i: (bi, i, 0)),
        out_shape=jax.ShapeDtypeStruct((b, n, 64), jnp.float32),
    )(tbl, g, wq, wkv, pb1, pw2, pb2, aw1, ab1, aw2, ab2, fw, fb)


def kernel(x, pos, wq, wk, wv, pw1, pb1, pw2, pb2, aw1, ab1, aw2, ab2, fw, fb):
    b, n, dim = x.shape
    tot = b * n * K

    post = jnp.transpose(pos, (0, 2, 1))
    wkv = jnp.concatenate([wk, wv], axis=1)
    r1 = lambda v: v.reshape(1, -1)

    knn, tbl = _knn(pos, post, x, pw1)
    info = plsc.get_sparse_core_info()
    nw = info.num_cores * info.num_subcores
    tbl2 = tbl.reshape(b * n, TW)
    outs = []
    for bi in range(b):
        idx3 = knn[bi].reshape(nw, n * K // (nw * CHUNK), CHUNK)
        g = _sc_gather(tbl2, idx3, n * K)
        outs.append(_attn(tbl[bi:bi + 1], g.reshape(1, n * K, TW), wq, wkv,
                          r1(pb1), pw2, r1(pb2), aw1, r1(ab1), aw2, r1(ab2),
                          fw, r1(fb)))
    return jnp.concatenate(outs, axis=0)


# speedup vs baseline: 25.1691x; 1.0134x over previous
import functools

import jax
import jax.numpy as jnp
from jax import lax
from jax.experimental import pallas as pl
from jax.experimental.pallas import tpu as pltpu
from jax.experimental.pallas import tpu_sc as plsc

K = 16
TW = 128
RB = 256
BN = 256
CHUNK = 128
NBUF = 4


SEG = 256
DPT = 3


def _topk_cols(vals, idxs, n_total):
    cols = []
    last = None
    for _ in range(K):
        m = jnp.min(vals, axis=1, keepdims=True)
        hit = vals <= m
        cols.append(jnp.min(jnp.where(hit, idxs, n_total), axis=1,
                            keepdims=True))
        vals = jnp.where(hit, jnp.inf, vals)
        last = m
    return jnp.concatenate(cols, axis=1), last


def _dists(pos_ref, post_ref):
    n_total = post_ref.shape[2]
    pc = pos_ref[0]
    pt = post_ref[0]
    acc = jnp.zeros((pc.shape[0], n_total), jnp.float32)
    for d in range(3):
        diff = pc[:, d:d + 1] - pt[d:d + 1, :]
        acc = acc + diff * diff
    return acc


def _knn_body(pos_ref, post_ref, x_ref, pw1_ref, out_ref, tbl_ref):
    b = pl.program_id(0)
    n_total = post_ref.shape[2]
    tbl_ref[0] = jnp.concatenate(
        [x_ref[0], _dot(pos_ref[0], pw1_ref[...])], axis=1)
    acc = _dists(pos_ref, post_ref)
    iota = lax.broadcasted_iota(jnp.int32, acc.shape, 1)
    rows = acc.shape[0]
    a3 = acc.reshape(rows, n_total // SEG, SEG)
    i3 = iota.reshape(rows, n_total // SEG, SEG)
    cvals, cidxs = [], []
    cur = a3
    for _ in range(DPT):
        m = jnp.min(cur, axis=1)
        hit = cur <= m[:, None, :]
        mi = jnp.min(jnp.where(hit, i3, n_total), axis=1)
        cvals.append(m)
        cidxs.append(mi)
        cur = jnp.where(i3 == mi[:, None, :], jnp.inf, cur)
    m_next = jnp.min(cur, axis=1)
    cand = jnp.concatenate(cvals, axis=1)
    cidx = jnp.concatenate(cidxs, axis=1)
    cols, last = _topk_cols(cand, cidx, n_total)
    out_ref[0] = cols + b * n_total
    bad = jnp.max(jnp.where(m_next <= last, 1, 0))

    @pl.when(bad > 0)
    def _():
        acc2 = _dists(pos_ref, post_ref)
        cols2, _ = _topk_cols(acc2,
                              lax.broadcasted_iota(jnp.int32, acc2.shape, 1),
                              n_total)
        out_ref[0] = cols2 + b * n_total


def _knn(pos, post, x, pw1):
    b, n, dim = x.shape
    return pl.pallas_call(
        _knn_body,
        grid=(b, n // RB),
        in_specs=[
            pl.BlockSpec((1, RB, 3), lambda bi, i: (bi, i, 0)),
            pl.BlockSpec((1, 3, n), lambda bi, i: (bi, 0, 0)),
            pl.BlockSpec((1, RB, dim), lambda bi, i: (bi, i, 0)),
            pl.BlockSpec((3, dim), lambda bi, i: (0, 0)),
        ],
        out_specs=(
            pl.BlockSpec((1, RB, K), lambda bi, i: (bi, i, 0)),
            pl.BlockSpec((1, RB, TW), lambda bi, i: (bi, i, 0)),
        ),
        out_shape=(
            jax.ShapeDtypeStruct((b, n, K), jnp.int32),
            jax.ShapeDtypeStruct((b, n, TW), jnp.float32),
        ),
    )(pos, post, x, pw1)


def _sc_gather(tbl2, idx3, tot):
    info = plsc.get_sparse_core_info()
    nc, ns = info.num_cores, info.num_subcores
    nw = nc * ns
    cpw = idx3.shape[1]
    mesh = plsc.VectorSubcoreMesh(core_axis_name="c", subcore_axis_name="s")

    @functools.partial(
        pl.kernel,
        mesh=mesh,
        out_type=jax.ShapeDtypeStruct((tot, TW), jnp.float32),
        scratch_types=(
            [pltpu.VMEM((cpw, CHUNK), jnp.int32)]
            + [pltpu.VMEM((CHUNK, TW), jnp.float32) for _ in range(NBUF)]
            + [pltpu.SemaphoreType.DMA, pltpu.SemaphoreType.DMA]
        ),
    )
    def k(tbl_hbm, idx_hbm, out_hbm, idx_v, *rest):
        bufs = rest[:NBUF]
        gsem, ssem = rest[NBUF], rest[NBUF + 1]
        wid = lax.axis_index("s") * nc + lax.axis_index("c")
        pltpu.sync_copy(idx_hbm.at[wid], idx_v)
        for r in range(cpw // NBUF):
            gets = []
            for j in range(NBUF):
                c = r * NBUF + j
                gets.append(
                    pltpu.async_copy(tbl_hbm.at[idx_v.at[c]], bufs[j], gsem))
            for g in gets:
                g.wait()
            puts = []
            for j in range(NBUF):
                c = r * NBUF + j
                base = (wid * cpw + c) * CHUNK
                puts.append(
                    pltpu.async_copy(bufs[j], out_hbm.at[pl.ds(base, CHUNK)],
                                     ssem))
            for p in puts:
                p.wait()

    return k(tbl2, idx3)


def _dot(a, b):
    return jnp.dot(a, b, precision=lax.Precision.DEFAULT,
                   preferred_element_type=jnp.float32)


def _attn_body(tbl_ref, g_ref, wq_ref, wkv_ref, pb1_ref, pw2_ref,
               pb2_ref, aw1_ref, ab1_ref, aw2_ref, ab2_ref, fw_ref, fb_ref,
               out_ref):
    tbl = tbl_ref[0]
    g = g_ref[0]
    xc = tbl[:, 0:64]
    pic = tbl[:, 64:128]
    xg = g[:, 0:64]
    pjg = g[:, 64:128]

    q = _dot(xc, wq_ref[...])
    kv = _dot(xg, wkv_ref[...])
    pre = pic[:, None, :] - pjg.reshape(BN, K, 64) + pb1_ref[...]
    pe = _dot(jax.nn.relu(pre.reshape(BN * K, 64)),
              pw2_ref[...]) + pb2_ref[...]
    h = (q[:, None, :] - kv[:, 0:64].reshape(BN, K, 64)
         + pe.reshape(BN, K, 64))
    a = _dot(jax.nn.relu(_dot(h.reshape(BN * K, 64), aw1_ref[...])
                         + ab1_ref[...]), aw2_ref[...]) + ab2_ref[...]
    s = a.reshape(BN, K, 64) * 0.125
    s = s - jnp.max(s, axis=1, keepdims=True)
    e = jnp.exp(s)
    w = e / jnp.sum(e, axis=1, keepdims=True)
    vpe = kv[:, 64:128].reshape(BN, K, 64) + pe.reshape(BN, K, 64)
    agg = jnp.sum(w * vpe, axis=1)
    out_ref[0] = _dot(agg, fw_ref[...]) + fb_ref[...] + xc


def _attn(tbl, g, wq, wkv, pb1, pw2, pb2, aw1, ab1, aw2, ab2, fw, fb):
    b, n, _ = tbl.shape
    full = lambda s: pl.BlockSpec(s, lambda bi, i: tuple(0 for _ in s))
    return pl.pallas_call(
        _attn_body,
        grid=(b, n // BN),
        in_specs=[
            pl.BlockSpec((1, BN, TW), lambda bi, i: (bi, i, 0)),
            pl.BlockSpec((1, BN * K, TW), lambda bi, i: (bi, i, 0)),
            full((64, 64)), full((64, 128)), full((1, 64)),
            full((64, 64)), full((1, 64)), full((64, 64)), full((1, 64)),
            full((64, 64)), full((1, 64)), full((64, 64)), full((1, 64)),
        ],
        out_specs=pl.BlockSpec((1, BN, 64), lambda bi, i: (bi, i, 0)),
        out_shape=jax.ShapeDtypeStruct((b, n, 64), jnp.float32),
    )(tbl, g, wq, wkv, pb1, pw2, pb2, aw1, ab1, aw2, ab2, fw, fb)


def kernel(x, pos, wq, wk, wv, pw1, pb1, pw2, pb2, aw1, ab1, aw2, ab2, fw, fb):
    b, n, dim = x.shape
    tot = b * n * K

    post = jnp.transpose(pos, (0, 2, 1))
    wkv = jnp.concatenate([wk, wv], axis=1)
    r1 = lambda v: v.reshape(1, -1)

    knn, tbl = _knn(pos, post, x, pw1)
    info = plsc.get_sparse_core_info()
    nw = info.num_cores * info.num_subcores
    tbl2 = tbl.reshape(b * n, TW)
    outs = []
    for bi in range(b):
        idx3 = knn[bi].reshape(nw, n * K // (nw * CHUNK), CHUNK)
        g = _sc_gather(tbl2, idx3, n * K)
        outs.append(_attn(tbl[bi:bi + 1], g.reshape(1, n * K, TW), wq, wkv,
                          r1(pb1), pw2, r1(pb2), aw1, r1(ab1), aw2, r1(ab2),
                          fw, r1(fb)))
    return jnp.concatenate(outs, axis=0)
